# baseline (speedup 1.0000x reference)
import functools

import jax
import jax.numpy as jnp
from jax import lax
from jax.experimental import pallas as pl
from jax.experimental.pallas import tpu as pltpu
from jax.experimental.pallas import tpu_sc as plsc

_N = 65536
_ALPHA = 50.0
_RGB_W = 1.0
_MASK_W = 2.0
_EIK_W = 0.1
_CSDF_W = 1.0
_CREG_W = 1.0

_NC = 2
_NS = 16
_NW = _NC * _NS
_L = 16
_R = _N // _NW
_CH = _R // _L


def _rsqrt(s):
    i = plsc.bitcast(s, jnp.int32)
    i = jnp.int32(0x5F3759DF) - (i >> 1)
    y = plsc.bitcast(i, jnp.float32)
    for _ in range(3):
        y = y * (1.5 - 0.5 * s * y * y)
    return y


def _softplus_neg(a):
    u = jnp.exp(-a)
    s = u / (2.0 + u)
    s2 = s * s
    return 2.0 * s * (1.0 + s2 * (1.0 / 3.0 + s2 * (
        1.0 / 5.0 + s2 * (1.0 / 7.0 + s2 * (1.0 / 9.0)))))


def _sc_body(rgb_a, rgb_b, pm, gm, sdf, grad, sh, sd, nr, out,
             rgb_a_v, rgb_b_v, pm_v, gm_v, sdf_v, grad_v, sh_v, sd_v, nr_v,
             part_v, sem):
    wid = lax.axis_index("s") * _NC + lax.axis_index("c")
    base = wid * _R
    base3 = base * 3

    copies = [
        pltpu.async_copy(rgb_a.at[pl.ds(base3, _R * 3)], rgb_a_v, sem),
        pltpu.async_copy(rgb_b.at[pl.ds(base3, _R * 3)], rgb_b_v, sem),
        pltpu.async_copy(pm.at[pl.ds(base, _R)], pm_v, sem),
        pltpu.async_copy(gm.at[pl.ds(base, _R)], gm_v, sem),
        pltpu.async_copy(sdf.at[pl.ds(base, _R)], sdf_v, sem),
        pltpu.async_copy(grad.at[pl.ds(base3, _R * 3)], grad_v, sem),
        pltpu.async_copy(sh.at[pl.ds(base, _R)], sh_v, sem),
        pltpu.async_copy(sd.at[pl.ds(base, _R)], sd_v, sem),
        pltpu.async_copy(nr.at[pl.ds(base3, _R * 3)], nr_v, sem),
    ]
    for c in copies:
        c.wait()

    iota3 = lax.iota(jnp.int32, _L) * 3
    zero = jnp.zeros((_L,), jnp.float32)

    def body(i, accs):
        a0, a1, a2, a3, a4, a5 = accs
        off = i * _L
        sl = pl.ds(off, _L)
        r0 = iota3 + off * 3
        r1 = r0 + 1
        r2 = r0 + 2

        pmv = pm_v[sl]
        gmv = gm_v[sl]
        m = pmv * gmv

        ax = plsc.load_gather(rgb_a_v, [r0])
        ay = plsc.load_gather(rgb_a_v, [r1])
        az = plsc.load_gather(rgb_a_v, [r2])
        bx = plsc.load_gather(rgb_b_v, [r0])
        by = plsc.load_gather(rgb_b_v, [r1])
        bz = plsc.load_gather(rgb_b_v, [r2])
        d = jnp.abs(ax - bx) + jnp.abs(ay - by) + jnp.abs(az - bz)
        a0 = a0 + d * m

        z = -_ALPHA * sdf_v[sl]
        bce = jnp.maximum(z, 0.0) - z * gmv + _softplus_neg(jnp.abs(z))
        a1 = a1 + bce * (1.0 - m)

        gx = plsc.load_gather(grad_v, [r0])
        gy = plsc.load_gather(grad_v, [r1])
        gz = plsc.load_gather(grad_v, [r2])
        s = gx * gx + gy * gy + gz * gz
        ns = s * _rsqrt(jnp.maximum(s, 1e-30))
        t = ns - 1.0
        a2 = a2 + t * t

        shv = sh_v[sl]
        sdv = sd_v[sl]
        cm = jnp.where((shv < 0.0) & (sdv < 0.0), 1.0, 0.0)
        a3 = a3 + jnp.maximum(-shv, 0.0) * cm
        a4 = a4 + cm

        nx = plsc.load_gather(nr_v, [r0])
        ny = plsc.load_gather(nr_v, [r1])
        nz = plsc.load_gather(nr_v, [r2])
        a5 = a5 + (nx * nx + ny * ny + nz * nz) * (1.0 - cm)

        return (a0, a1, a2, a3, a4, a5)

    accs = lax.fori_loop(0, _CH, body, (zero, zero, zero, zero, zero, zero))
    for k in range(6):
        part_v[k, :] = accs[k]
    pltpu.sync_copy(part_v, out.at[wid])


_sc_partials = functools.partial(
    pl.kernel,
    mesh=plsc.VectorSubcoreMesh(core_axis_name="c", subcore_axis_name="s"),
    out_type=jax.ShapeDtypeStruct((_NW, 6, _L), jnp.float32),
    compiler_params=pltpu.CompilerParams(
        needs_layout_passes=False,
        skip_device_barrier=True,
    ),
    scratch_types=[
        pltpu.VMEM((_R * 3,), jnp.float32),
        pltpu.VMEM((_R * 3,), jnp.float32),
        pltpu.VMEM((_R,), jnp.float32),
        pltpu.VMEM((_R,), jnp.float32),
        pltpu.VMEM((_R,), jnp.float32),
        pltpu.VMEM((_R * 3,), jnp.float32),
        pltpu.VMEM((_R,), jnp.float32),
        pltpu.VMEM((_R,), jnp.float32),
        pltpu.VMEM((_R * 3,), jnp.float32),
        pltpu.VMEM((6, _L), jnp.float32),
        pltpu.SemaphoreType.DMA,
    ],
)(_sc_body)


def _fin_body(x_ref, o_ref):
    x = x_ref[...]
    p = [jnp.sum(x[:, k * _L:(k + 1) * _L]) for k in range(6)]
    n = float(_N)
    rgb_loss = p[0] / n
    mask_loss = (1.0 / _ALPHA) * p[1] / n
    eik_loss = p[2] / n
    contact_loss = p[3] / jnp.maximum(p[4], 1.0)
    contact_reg = p[5] / jnp.maximum((n - p[4]) * 3.0, 1.0)
    o_ref[0, 0] = (_RGB_W * rgb_loss + _MASK_W * mask_loss +
                   _EIK_W * eik_loss + _CSDF_W * contact_loss +
                   _CREG_W * contact_reg)


_finalize = pl.pallas_call(
    _fin_body,
    out_shape=jax.ShapeDtypeStruct((1, 1), jnp.float32),
    out_specs=pl.BlockSpec(memory_space=pltpu.SMEM),
)


@jax.jit
def kernel(rgb_values, rgb_gt, pred_mask, gt_mask, sdf_output, grad_theta,
           sdf_head, sdf_hand, nonrigid_deformation):
    pm = pred_mask.astype(jnp.float32)
    gm = gt_mask.astype(jnp.float32)
    sdf = sdf_output.reshape(-1)
    parts = _sc_partials(rgb_values.reshape(-1), rgb_gt.reshape(-1), pm, gm,
                         sdf, grad_theta.reshape(-1), sdf_head, sdf_hand,
                         nonrigid_deformation.reshape(-1))
    total = _finalize(parts.reshape(_NW, 6 * _L))
    return total[0, 0]


# speedup vs baseline: 5.7221x; 5.7221x over previous
import functools

import jax
import jax.numpy as jnp
from jax import lax
from jax.experimental import pallas as pl
from jax.experimental.pallas import tpu as pltpu
from jax.experimental.pallas import tpu_sc as plsc

_N = 65536
_ALPHA = 50.0
_RGB_W = 1.0
_MASK_W = 2.0
_EIK_W = 0.1
_CSDF_W = 1.0
_CREG_W = 1.0

_NC = 2
_NS = 16
_NW = _NC * _NS
_L = 16
_R = _N // _NW
_CH = _R // _L


def _rsqrt(s):
    i = plsc.bitcast(s, jnp.int32)
    i = jnp.int32(0x5F3759DF) - (i >> 1)
    y = plsc.bitcast(i, jnp.float32)
    for _ in range(3):
        y = y * (1.5 - 0.5 * s * y * y)
    return y


def _softplus_neg(a):
    u = jnp.exp(-a)
    s = u / (2.0 + u)
    s2 = s * s
    return 2.0 * s * (1.0 + s2 * (1.0 / 3.0 + s2 * (
        1.0 / 5.0 + s2 * (1.0 / 7.0 + s2 * (1.0 / 9.0)))))


def _sc_body(rgb_a, rgb_b, pm, gm, sdf, grad, sh, sd, nr, out,
             ax_v, ay_v, az_v, bx_v, by_v, bz_v,
             gx_v, gy_v, gz_v, nx_v, ny_v, nz_v,
             pm_v, gm_v, sdf_v, sh_v, sd_v,
             part_v, sem):
    wid = lax.axis_index("s") * _NC + lax.axis_index("c")
    base = wid * _R

    copies = [
        pltpu.async_copy(rgb_a.at[pl.ds(base, _R)], ax_v, sem),
        pltpu.async_copy(rgb_a.at[pl.ds(_N + base, _R)], ay_v, sem),
        pltpu.async_copy(rgb_a.at[pl.ds(2 * _N + base, _R)], az_v, sem),
        pltpu.async_copy(rgb_b.at[pl.ds(base, _R)], bx_v, sem),
        pltpu.async_copy(rgb_b.at[pl.ds(_N + base, _R)], by_v, sem),
        pltpu.async_copy(rgb_b.at[pl.ds(2 * _N + base, _R)], bz_v, sem),
        pltpu.async_copy(grad.at[pl.ds(base, _R)], gx_v, sem),
        pltpu.async_copy(grad.at[pl.ds(_N + base, _R)], gy_v, sem),
        pltpu.async_copy(grad.at[pl.ds(2 * _N + base, _R)], gz_v, sem),
        pltpu.async_copy(nr.at[pl.ds(base, _R)], nx_v, sem),
        pltpu.async_copy(nr.at[pl.ds(_N + base, _R)], ny_v, sem),
        pltpu.async_copy(nr.at[pl.ds(2 * _N + base, _R)], nz_v, sem),
        pltpu.async_copy(pm.at[pl.ds(base, _R)], pm_v, sem),
        pltpu.async_copy(gm.at[pl.ds(base, _R)], gm_v, sem),
        pltpu.async_copy(sdf.at[pl.ds(base, _R)], sdf_v, sem),
        pltpu.async_copy(sh.at[pl.ds(base, _R)], sh_v, sem),
        pltpu.async_copy(sd.at[pl.ds(base, _R)], sd_v, sem),
    ]
    for c in copies:
        c.wait()

    zero = jnp.zeros((_L,), jnp.float32)

    def body(i, accs):
        a0, a1, a2, a3, a4, a5 = accs
        sl = pl.ds(i * _L, _L)

        pmv = pm_v[sl]
        gmv = gm_v[sl]
        m = pmv * gmv

        d = (jnp.abs(ax_v[sl] - bx_v[sl]) + jnp.abs(ay_v[sl] - by_v[sl]) +
             jnp.abs(az_v[sl] - bz_v[sl]))
        a0 = a0 + d * m

        z = -_ALPHA * sdf_v[sl]
        bce = jnp.maximum(z, 0.0) - z * gmv + _softplus_neg(jnp.abs(z))
        a1 = a1 + bce * (1.0 - m)

        gx = gx_v[sl]
        gy = gy_v[sl]
        gz = gz_v[sl]
        s = gx * gx + gy * gy + gz * gz
        ns = s * _rsqrt(jnp.maximum(s, 1e-30))
        t = ns - 1.0
        a2 = a2 + t * t

        shv = sh_v[sl]
        sdv = sd_v[sl]
        cm = jnp.where((shv < 0.0) & (sdv < 0.0), 1.0, 0.0)
        a3 = a3 + jnp.maximum(-shv, 0.0) * cm
        a4 = a4 + cm

        nx = nx_v[sl]
        ny = ny_v[sl]
        nz = nz_v[sl]
        a5 = a5 + (nx * nx + ny * ny + nz * nz) * (1.0 - cm)

        return (a0, a1, a2, a3, a4, a5)

    accs = lax.fori_loop(0, _CH, body, (zero, zero, zero, zero, zero, zero))
    for k in range(6):
        part_v[k, :] = accs[k]
    pltpu.sync_copy(part_v, out.at[wid])


_sc_partials = functools.partial(
    pl.kernel,
    mesh=plsc.VectorSubcoreMesh(core_axis_name="c", subcore_axis_name="s"),
    out_type=jax.ShapeDtypeStruct((_NW, 6, _L), jnp.float32),
    compiler_params=pltpu.CompilerParams(
        needs_layout_passes=False,
        skip_device_barrier=True,
    ),
    scratch_types=(
        [pltpu.VMEM((_R,), jnp.float32) for _ in range(17)] +
        [pltpu.VMEM((6, _L), jnp.float32), pltpu.SemaphoreType.DMA]
    ),
)(_sc_body)


def _fin_body(x_ref, o_ref):
    x = x_ref[...]
    p = [jnp.sum(x[:, k * _L:(k + 1) * _L]) for k in range(6)]
    n = float(_N)
    rgb_loss = p[0] / n
    mask_loss = (1.0 / _ALPHA) * p[1] / n
    eik_loss = p[2] / n
    contact_loss = p[3] / jnp.maximum(p[4], 1.0)
    contact_reg = p[5] / jnp.maximum((n - p[4]) * 3.0, 1.0)
    o_ref[0, 0] = (_RGB_W * rgb_loss + _MASK_W * mask_loss +
                   _EIK_W * eik_loss + _CSDF_W * contact_loss +
                   _CREG_W * contact_reg)


_finalize = pl.pallas_call(
    _fin_body,
    out_shape=jax.ShapeDtypeStruct((1, 1), jnp.float32),
    out_specs=pl.BlockSpec(memory_space=pltpu.SMEM),
)


@jax.jit
def kernel(rgb_values, rgb_gt, pred_mask, gt_mask, sdf_output, grad_theta,
           sdf_head, sdf_hand, nonrigid_deformation):
    pm = pred_mask.astype(jnp.float32)
    gm = gt_mask.astype(jnp.float32)
    sdf = sdf_output.reshape(-1)
    parts = _sc_partials(rgb_values.T.reshape(-1), rgb_gt.T.reshape(-1),
                         pm, gm, sdf, grad_theta.T.reshape(-1),
                         sdf_head, sdf_hand,
                         nonrigid_deformation.T.reshape(-1))
    total = _finalize(parts.reshape(_NW, 6 * _L))
    return total[0, 0]
